# Initial kernel scaffold; baseline (speedup 1.0000x reference)
import functools

import jax
import jax.numpy as jnp
from jax import lax
from jax.experimental import pallas as pl
from jax.experimental.pallas import tpu as pltpu
from jax.experimental.pallas import tpu_sc as plsc

N = 10000
E = 320000
D = 128
H = 128
L = 128
G = 64
NLAYERS = 3

NC = 2
NS = 16
NW = NC * NS

C = 128
EW = -(-E // NW)
CPT = -(-EW // C)
EWP = CPT * C
EP = NW * EWP

NPAD = 10016
RPT = NPAD // NS


def _make_seg_sum():
  mesh = plsc.VectorSubcoreMesh(core_axis_name="c", subcore_axis_name="s")

  @functools.partial(
      pl.kernel,
      out_type=jax.ShapeDtypeStruct((NC * NPAD, D), jnp.float32),
      mesh=mesh,
      scratch_types=[
          pltpu.VMEM((C,), jnp.int32),
          pltpu.VMEM((C,), jnp.int32),
          pltpu.VMEM((C, D), jnp.float32),
          pltpu.VMEM_SHARED((NPAD, D), jnp.float32),
          pltpu.SemaphoreType.DMA,
      ],
  )
  def seg_sum(h_hbm, src_hbm, dst_hbm, zeros_hbm, out_hbm,
              src_v, dst_v, rows_v, acc_sh, sem):
    c = lax.axis_index("c")
    s = lax.axis_index("s")
    wid = c * NS + s
    base_e = wid * EWP
    r0 = s * RPT

    pltpu.sync_copy(zeros_hbm.at[pl.ds(r0, RPT)], acc_sh.at[pl.ds(r0, RPT)])
    plsc.subcore_barrier()

    def body(i, carry):
      base = base_e + i * C
      pltpu.sync_copy(src_hbm.at[pl.ds(base, C)], src_v)
      pltpu.sync_copy(dst_hbm.at[pl.ds(base, C)], dst_v)
      pltpu.async_copy(h_hbm.at[src_v], rows_v, sem).wait()
      pltpu.sync_copy(rows_v, acc_sh.at[dst_v], add=True)
      return carry

    lax.fori_loop(0, CPT, body, 0)
    plsc.subcore_barrier()
    pltpu.sync_copy(acc_sh.at[pl.ds(r0, RPT)],
                    out_hbm.at[pl.ds(c * NPAD + r0, RPT)])

  return seg_sum


_seg_sum = _make_seg_sum()


def _leaky(x):
  return jnp.where(x >= 0, x, 0.2 * x)


def _layer_body(parts_ref, h_ref, w1_ref, b1_ref, g_ref, be_ref, w2_ref,
                b2_ref, o_ref):
  agg = parts_ref[0:N, :] + parts_ref[NPAD:NPAD + N, :]
  x = agg + h_ref[...]
  t = jnp.dot(x, w1_ref[...], preferred_element_type=jnp.float32)
  t = _leaky(t + b1_ref[...])
  mu = jnp.mean(t, axis=0, keepdims=True)
  var = jnp.mean((t - mu) * (t - mu), axis=0, keepdims=True)
  t = (t - mu) * lax.rsqrt(var + 1e-5) * g_ref[...] + be_ref[...]
  t = jnp.dot(t, w2_ref[...], preferred_element_type=jnp.float32)
  o_ref[...] = _leaky(t + b2_ref[...])


_layer_tc = pl.pallas_call(
    _layer_body,
    out_shape=jax.ShapeDtypeStruct((N, H), jnp.float32),
)


def _final_body(h_ref, batch_ref, wa_ref, ba_ref, gg_ref, beg_ref, wl_ref,
                bl_ref, o_ref):
  h = h_ref[...]
  s = jnp.sum(h * wa_ref[...], axis=1, keepdims=True) + ba_ref[...]
  att = jax.nn.sigmoid(s)
  hw = h * att
  gid = lax.broadcasted_iota(jnp.int32, (G, N), 0)
  m = (gid == batch_ref[...]).astype(jnp.float32)
  pooled = jnp.dot(m, hw, preferred_element_type=jnp.float32)
  mu = jnp.mean(pooled, axis=0, keepdims=True)
  var = jnp.mean((pooled - mu) * (pooled - mu), axis=0, keepdims=True)
  nrm = (pooled - mu) * lax.rsqrt(var + 1e-5) * gg_ref[...] + beg_ref[...]
  o_ref[...] = jnp.dot(nrm, wl_ref[...],
                       preferred_element_type=jnp.float32) + bl_ref[...]


_final_tc = pl.pallas_call(
    _final_body,
    out_shape=jax.ShapeDtypeStruct((G, L), jnp.float32),
)


def kernel(x, edge_index, batch, params):
  src = edge_index[0]
  dst = edge_index[1]
  pad = EP - E
  src_p = jnp.concatenate([src, jnp.zeros((pad,), jnp.int32)])
  dst_p = jnp.concatenate([dst, jnp.full((pad,), N, jnp.int32)])
  zeros = jnp.zeros((NPAD, D), jnp.float32)
  batch2d = batch.reshape(1, N)

  h = x
  for i in range(NLAYERS):
    parts = _seg_sum(h, src_p, dst_p, zeros)
    h = _layer_tc(
        parts, h,
        params['W1_%d' % i], params['b1_%d' % i].reshape(1, H),
        params['g_%d' % i].reshape(1, H), params['be_%d' % i].reshape(1, H),
        params['W2_%d' % i], params['b2_%d' % i].reshape(1, H))

  out = _final_tc(
      h, batch2d, params['Wa'].reshape(1, H), params['ba'].reshape(1, 1),
      params['g_glob'].reshape(1, H), params['be_glob'].reshape(1, H),
      params['Wl'], params['bl'].reshape(1, L))
  return (h, out)


# speedup vs baseline: 3.5024x; 3.5024x over previous
import functools

import jax
import jax.numpy as jnp
from jax import lax
from jax.experimental import pallas as pl
from jax.experimental.pallas import tpu as pltpu
from jax.experimental.pallas import tpu_sc as plsc

N = 10000
E = 320000
D = 128
H = 128
L = 128
G = 64
NLAYERS = 3

NC = 2
NS = 16
NW = NC * NS

C = 128
EW = -(-E // NW)
CPT = -(-EW // C)
EWP = CPT * C
EP = NW * EWP

NPAD = 10112
RPT = NPAD // NS


@functools.cache
def _make_seg_sum():
  mesh = plsc.VectorSubcoreMesh(core_axis_name="c", subcore_axis_name="s",
                                num_cores=NC, num_subcores=NS)

  @functools.partial(
      pl.kernel,
      out_type=jax.ShapeDtypeStruct((NC * NPAD, D), jnp.float32),
      mesh=mesh,
      scratch_types=[
          pltpu.VMEM((C,), jnp.int32),
          pltpu.VMEM((C,), jnp.int32),
          pltpu.VMEM((C, D), jnp.float32),
          pltpu.VMEM_SHARED((NPAD, D), jnp.float32),
          pltpu.SemaphoreType.DMA,
      ],
  )
  def seg_sum(h_hbm, src_hbm, dst_hbm, zeros_hbm, out_hbm,
              src_v, dst_v, rows_v, acc_sh, sem):
    c = lax.axis_index("c")
    s = lax.axis_index("s")
    wid = c * NS + s
    base_e = wid * EWP
    r0 = s * RPT

    pltpu.sync_copy(zeros_hbm.at[pl.ds(r0, RPT)], acc_sh.at[pl.ds(r0, RPT)])
    plsc.subcore_barrier()

    def body(i, carry):
      base = base_e + i * C
      pltpu.sync_copy(src_hbm.at[pl.ds(base, C)], src_v)
      pltpu.sync_copy(dst_hbm.at[pl.ds(base, C)], dst_v)
      pltpu.async_copy(h_hbm.at[src_v], rows_v, sem).wait()
      pltpu.sync_copy(rows_v, acc_sh.at[dst_v], add=True)
      return carry

    lax.fori_loop(0, CPT, body, 0)
    plsc.subcore_barrier()
    pltpu.sync_copy(acc_sh.at[pl.ds(r0, RPT)],
                    out_hbm.at[pl.ds(c * NPAD + r0, RPT)])

  return seg_sum


def _leaky(x):
  return jnp.where(x >= 0, x, 0.2 * x)


def _layer_body(parts_ref, h_ref, w1_ref, b1_ref, g_ref, be_ref, w2_ref,
                b2_ref, o_ref):
  agg = parts_ref[0:N, :] + parts_ref[NPAD:NPAD + N, :]
  x = agg + h_ref[...]
  t = jnp.dot(x, w1_ref[...], preferred_element_type=jnp.float32)
  t = _leaky(t + b1_ref[...])
  mu = jnp.mean(t, axis=0, keepdims=True)
  var = jnp.mean((t - mu) * (t - mu), axis=0, keepdims=True)
  t = (t - mu) * lax.rsqrt(var + 1e-5) * g_ref[...] + be_ref[...]
  t = jnp.dot(t, w2_ref[...], preferred_element_type=jnp.float32)
  o_ref[...] = _leaky(t + b2_ref[...])


_layer_tc = pl.pallas_call(
    _layer_body,
    out_shape=jax.ShapeDtypeStruct((N, H), jnp.float32),
)


def _final_body(h_ref, batch_ref, wa_ref, ba_ref, gg_ref, beg_ref, wl_ref,
                bl_ref, o_ref):
  h = h_ref[...]
  s = jnp.sum(h * wa_ref[...], axis=1, keepdims=True) + ba_ref[...]
  att = jax.nn.sigmoid(s)
  hw = h * att
  gid = lax.broadcasted_iota(jnp.int32, (G, N), 0)
  m = (gid == batch_ref[...]).astype(jnp.float32)
  pooled = jnp.dot(m, hw, preferred_element_type=jnp.float32)
  mu = jnp.mean(pooled, axis=0, keepdims=True)
  var = jnp.mean((pooled - mu) * (pooled - mu), axis=0, keepdims=True)
  nrm = (pooled - mu) * lax.rsqrt(var + 1e-5) * gg_ref[...] + beg_ref[...]
  o_ref[...] = jnp.dot(nrm, wl_ref[...],
                       preferred_element_type=jnp.float32) + bl_ref[...]


_final_tc = pl.pallas_call(
    _final_body,
    out_shape=jax.ShapeDtypeStruct((G, L), jnp.float32),
)


def kernel(x, edge_index, batch, params):
  src = edge_index[0]
  dst = edge_index[1]
  pad = EP - E
  src_p = jnp.concatenate([src, jnp.zeros((pad,), jnp.int32)])
  dst_p = jnp.concatenate([dst, jnp.full((pad,), N, jnp.int32)])
  zeros = jnp.zeros((NPAD, D), jnp.float32)
  batch2d = batch.reshape(1, N)

  h = x
  for i in range(NLAYERS):
    parts = _make_seg_sum()(h, src_p, dst_p, zeros)
    h = _layer_tc(
        parts, h,
        params['W1_%d' % i], params['b1_%d' % i].reshape(1, H),
        params['g_%d' % i].reshape(1, H), params['be_%d' % i].reshape(1, H),
        params['W2_%d' % i], params['b2_%d' % i].reshape(1, H))

  out = _final_tc(
      h, batch2d, params['Wa'].reshape(1, H), params['ba'].reshape(1, 1),
      params['g_glob'].reshape(1, H), params['be_glob'].reshape(1, H),
      params['Wl'], params['bl'].reshape(1, L))
  return (h, out)
